# baseline (speedup 1.0000x reference)
import jax
import jax.numpy as jnp
from jax import lax
from jax.experimental import pallas as pl
from jax.experimental.pallas import tpu as pltpu
from jax.experimental.pallas import tpu_sc as plsc

N_CLASSES = 1000
CTX_LEN = 77
D_MODEL = 512

NC, NS = 2, 16
NW = NC * NS
B = N_CLASSES * CTX_LEN
K = 112
NCHUNK = 22
BPW = K * NCHUNK
BPAD = BPW * NW


def _gather_body(table_hbm, idx_hbm, out_hbm, idx_v, buf0, buf1, sem0, sem1,
                 osem):
    c = lax.axis_index("c")
    s = lax.axis_index("s")
    wid = s * NC + c
    base = wid * BPW
    pltpu.sync_copy(idx_hbm.at[pl.ds(base, BPW)], idx_v)

    bufs = (buf0, buf1)
    sems = (sem0, sem1)

    pltpu.async_copy(table_hbm.at[idx_v.at[pl.ds(0, K)]], bufs[0], sems[0])

    def body(g, carry):
        for b in range(2):
            i = g * 2 + b
            buf, sem = bufs[b], sems[b]
            nbuf, nsem = bufs[1 - b], sems[1 - b]
            pltpu.make_async_copy(table_hbm.at[idx_v.at[pl.ds(0, K)]], buf,
                                  sem).wait()

            @pl.when(i + 1 < NCHUNK)
            def _():
                off = (i + 1) * K
                pltpu.async_copy(table_hbm.at[idx_v.at[pl.ds(off, K)]], nbuf,
                                 nsem)

            pltpu.async_copy(buf, out_hbm.at[pl.ds(base + i * K, K)],
                             osem).wait()
        return carry

    lax.fori_loop(0, NCHUNK // 2, body, 0)


def _sc_gather(table, idx_pad):
    mesh = plsc.VectorSubcoreMesh(core_axis_name="c", subcore_axis_name="s")
    f = pl.kernel(
        _gather_body,
        out_type=jax.ShapeDtypeStruct((BPAD, D_MODEL), jnp.float32),
        mesh=mesh,
        scratch_types=[
            pltpu.VMEM((BPW,), jnp.int32),
            pltpu.VMEM((K, D_MODEL), jnp.float32),
            pltpu.VMEM((K, D_MODEL), jnp.float32),
            pltpu.SemaphoreType.DMA,
            pltpu.SemaphoreType.DMA,
            pltpu.SemaphoreType.DMA,
        ],
        name="sc_embedding_gather",
    )
    return f(table, idx_pad)


def _argmax_body(ids_ref, out_ref):
    ids = ids_ref[...]
    iota = lax.broadcasted_iota(jnp.int32, ids.shape, 1)
    m = jnp.max(ids, axis=1, keepdims=True)
    cand = jnp.where(ids == m, iota, CTX_LEN)
    out_ref[...] = jnp.min(cand, axis=1, keepdims=True)


def _tc_argmax(prompt):
    return pl.pallas_call(
        _argmax_body,
        out_shape=jax.ShapeDtypeStruct((N_CLASSES, 1), jnp.int32),
    )(prompt)


def kernel(prompt, table):
    idx = prompt.reshape(-1)
    idx_pad = jnp.pad(idx, (0, BPAD - B))
    rows = _sc_gather(table, idx_pad)
    embedding = rows[:B].reshape(N_CLASSES, CTX_LEN, D_MODEL)
    eos = _tc_argmax(prompt).reshape(N_CLASSES)
    return (embedding, eos)


# speedup vs baseline: 1.2034x; 1.2034x over previous
import jax
import jax.numpy as jnp
from jax import lax
from jax.experimental import pallas as pl
from jax.experimental.pallas import tpu as pltpu
from jax.experimental.pallas import tpu_sc as plsc

N_CLASSES = 1000
CTX_LEN = 77
D_MODEL = 512

NC, NS = 2, 16
NW = NC * NS
B = N_CLASSES * CTX_LEN
K = 112
NCHUNK = 22
BPW = K * NCHUNK
STRIDE = 2408


def _gather_body(table_hbm, idx_hbm, out_hbm, idx_v, buf0, buf1, sem0, sem1,
                 osem):
    c = lax.axis_index("c")
    s = lax.axis_index("s")
    wid = s * NC + c
    base = pl.multiple_of(jnp.minimum(wid * STRIDE, B - BPW), 8)
    pltpu.sync_copy(idx_hbm.at[pl.ds(base, BPW)], idx_v)

    bufs = (buf0, buf1)
    sems = (sem0, sem1)

    pltpu.async_copy(table_hbm.at[idx_v.at[pl.ds(0, K)]], bufs[0], sems[0])

    def body(g, carry):
        for b in range(2):
            i = g * 2 + b
            buf, sem = bufs[b], sems[b]
            nbuf, nsem = bufs[1 - b], sems[1 - b]
            pltpu.make_async_copy(table_hbm.at[idx_v.at[pl.ds(0, K)]], buf,
                                  sem).wait()

            @pl.when(i + 1 < NCHUNK)
            def _():
                off = (i + 1) * K
                pltpu.async_copy(table_hbm.at[idx_v.at[pl.ds(off, K)]], nbuf,
                                 nsem)

            pltpu.async_copy(buf, out_hbm.at[pl.ds(base + i * K, K)],
                             osem).wait()
        return carry

    lax.fori_loop(0, NCHUNK // 2, body, 0)


def _sc_gather(table, idx_pad):
    mesh = plsc.VectorSubcoreMesh(core_axis_name="c", subcore_axis_name="s")
    f = pl.kernel(
        _gather_body,
        out_type=jax.ShapeDtypeStruct((B, D_MODEL), jnp.float32),
        mesh=mesh,
        scratch_types=[
            pltpu.VMEM((BPW,), jnp.int32),
            pltpu.VMEM((K, D_MODEL), jnp.float32),
            pltpu.VMEM((K, D_MODEL), jnp.float32),
            pltpu.SemaphoreType.DMA,
            pltpu.SemaphoreType.DMA,
            pltpu.SemaphoreType.DMA,
        ],
        name="sc_embedding_gather",
    )
    return f(table, idx_pad)


def _argmax_body(ids_ref, out_ref):
    ids = ids_ref[...]
    iota = lax.broadcasted_iota(jnp.int32, ids.shape, 1)
    m = jnp.max(ids, axis=1, keepdims=True)
    cand = jnp.where(ids == m, iota, CTX_LEN)
    out_ref[...] = jnp.min(cand, axis=1, keepdims=True)


def _tc_argmax(prompt):
    return pl.pallas_call(
        _argmax_body,
        out_shape=jax.ShapeDtypeStruct((N_CLASSES, 1), jnp.int32),
    )(prompt)


def kernel(prompt, table):
    idx = prompt.reshape(-1)
    rows = _sc_gather(table, idx)
    embedding = rows.reshape(N_CLASSES, CTX_LEN, D_MODEL)
    eos = _tc_argmax(prompt).reshape(N_CLASSES)
    return (embedding, eos)


# speedup vs baseline: 5.2752x; 4.3834x over previous
import jax
import jax.numpy as jnp
from jax import lax
from jax.experimental import pallas as pl
from jax.experimental.pallas import tpu as pltpu
from jax.experimental.pallas import tpu_sc as plsc

N_CLASSES = 1000
CTX_LEN = 77
D_MODEL = 512

NC, NS = 2, 16
NW = NC * NS
B = N_CLASSES * CTX_LEN
K = 112
NCHUNK = 22
BPW = K * NCHUNK
STRIDE = 2408


def _gather_body(table_hbm, idx_hbm, out_hbm, idx_v, buf0, buf1, sem0, sem1,
                 osem):
    c = lax.axis_index("c")
    s = lax.axis_index("s")
    wid = s * NC + c
    base = pl.multiple_of(jnp.minimum(wid * STRIDE, B - BPW), 8)
    pltpu.sync_copy(idx_hbm.at[pl.ds(base, BPW)], idx_v)

    bufs = (buf0, buf1)
    sems = (sem0, sem1)

    pltpu.async_copy(table_hbm.at[idx_v.at[pl.ds(0, K)]], bufs[0], sems[0])

    def body(g, carry):
        for b in range(2):
            i = g * 2 + b
            buf, sem = bufs[b], sems[b]
            nbuf, nsem = bufs[1 - b], sems[1 - b]
            pltpu.make_async_copy(table_hbm.at[idx_v.at[pl.ds(0, K)]], buf,
                                  sem).wait()

            @pl.when(i + 1 < NCHUNK)
            def _():
                off = (i + 1) * K
                pltpu.async_copy(table_hbm.at[idx_v.at[pl.ds(off, K)]], nbuf,
                                 nsem)

            pltpu.async_copy(buf, out_hbm.at[pl.ds(base + i * K, K)],
                             osem).wait()
        return carry

    lax.fori_loop(0, NCHUNK // 2, body, 0)


def _sc_gather(table, idx_pad):
    mesh = plsc.VectorSubcoreMesh(core_axis_name="c", subcore_axis_name="s")
    f = pl.kernel(
        _gather_body,
        out_type=jax.ShapeDtypeStruct((B, D_MODEL), jnp.float32),
        mesh=mesh,
        scratch_types=[
            pltpu.VMEM((BPW,), jnp.int32),
            pltpu.VMEM((K, D_MODEL), jnp.float32),
            pltpu.VMEM((K, D_MODEL), jnp.float32),
            pltpu.SemaphoreType.DMA,
            pltpu.SemaphoreType.DMA,
            pltpu.SemaphoreType.DMA,
        ],
        name="sc_embedding_gather",
    )
    return f(table, idx_pad)


def _argmax_body(ids_ref, out_ref):
    ids = ids_ref[...]
    iota = lax.broadcasted_iota(jnp.int32, ids.shape, 1)
    m = jnp.max(ids, axis=1, keepdims=True)
    cand = jnp.where(ids == m, iota, CTX_LEN)
    out_ref[...] = jnp.min(cand, axis=1, keepdims=True)


def _tc_argmax(prompt):
    return pl.pallas_call(
        _argmax_body,
        out_shape=jax.ShapeDtypeStruct((N_CLASSES, 1), jnp.int32),
    )(prompt)


def kernel(prompt, table):
    idx = jnp.swapaxes(prompt, 0, 1).reshape(-1)
    rows = _sc_gather(table, idx)
    embedding = jnp.swapaxes(rows.reshape(CTX_LEN, N_CLASSES, D_MODEL), 0, 1)
    eos = _tc_argmax(prompt).reshape(N_CLASSES)
    return (embedding, eos)


# speedup vs baseline: 5.6375x; 1.0687x over previous
import jax
import jax.numpy as jnp
from jax import lax
from jax.experimental import pallas as pl
from jax.experimental.pallas import tpu as pltpu
from jax.experimental.pallas import tpu_sc as plsc

N_CLASSES = 1000
CTX_LEN = 77
D_MODEL = 512

NC, NS = 2, 16
NW = NC * NS
B = N_CLASSES * CTX_LEN
K = 80
NCHUNK = 31
BPW = K * NCHUNK
STRIDE = 2400


def _gather_body(table_hbm, idx_hbm, out_hbm, idx_v, buf0, buf1, buf2,
                 gsem0, gsem1, gsem2, wsem0, wsem1, wsem2):
    c = lax.axis_index("c")
    s = lax.axis_index("s")
    wid = s * NC + c
    base = pl.multiple_of(jnp.minimum(wid * STRIDE, B - BPW), 8)
    pltpu.sync_copy(idx_hbm.at[pl.ds(base, BPW)], idx_v)

    bufs = (buf0, buf1, buf2)
    gsems = (gsem0, gsem1, gsem2)
    wsems = (wsem0, wsem1, wsem2)

    def start_gather(i, b):
        pltpu.async_copy(table_hbm.at[idx_v.at[pl.ds(i * K, K)]], bufs[b],
                         gsems[b])

    start_gather(0, 0)
    start_gather(1, 1)

    def body(i, carry):
        slot = lax.rem(i, 3)
        for b in range(3):
            @pl.when(slot == b)
            def _(b=b):
                pltpu.make_async_copy(table_hbm.at[idx_v.at[pl.ds(0, K)]],
                                      bufs[b], gsems[b]).wait()
                pltpu.async_copy(bufs[b], out_hbm.at[pl.ds(base + i * K, K)],
                                 wsems[b])

                @pl.when(i + 2 < NCHUNK)
                def _():
                    nb = (b + 2) % 3

                    @pl.when(i >= 1)
                    def _():
                        pltpu.make_async_copy(
                            bufs[nb], out_hbm.at[pl.ds(base, K)],
                            wsems[nb]).wait()

                    start_gather(i + 2, nb)
        return carry

    lax.fori_loop(0, NCHUNK, body, 0)

    for i in (NCHUNK - 3, NCHUNK - 2, NCHUNK - 1):
        b = i % 3
        pltpu.make_async_copy(bufs[b], out_hbm.at[pl.ds(base, K)],
                              wsems[b]).wait()


def _sc_gather(table, idx_pad):
    mesh = plsc.VectorSubcoreMesh(core_axis_name="c", subcore_axis_name="s")
    f = pl.kernel(
        _gather_body,
        out_type=jax.ShapeDtypeStruct((B, D_MODEL), jnp.float32),
        mesh=mesh,
        scratch_types=[
            pltpu.VMEM((BPW,), jnp.int32),
            pltpu.VMEM((K, D_MODEL), jnp.float32),
            pltpu.VMEM((K, D_MODEL), jnp.float32),
            pltpu.VMEM((K, D_MODEL), jnp.float32),
            pltpu.SemaphoreType.DMA,
            pltpu.SemaphoreType.DMA,
            pltpu.SemaphoreType.DMA,
            pltpu.SemaphoreType.DMA,
            pltpu.SemaphoreType.DMA,
            pltpu.SemaphoreType.DMA,
        ],
        name="sc_embedding_gather",
    )
    return f(table, idx_pad)


def _argmax_body(ids_ref, out_ref):
    ids = ids_ref[...]
    iota = lax.broadcasted_iota(jnp.int32, ids.shape, 1)
    m = jnp.max(ids, axis=1, keepdims=True)
    cand = jnp.where(ids == m, iota, CTX_LEN)
    out_ref[...] = jnp.min(cand, axis=1, keepdims=True)


def _tc_argmax(prompt):
    return pl.pallas_call(
        _argmax_body,
        out_shape=jax.ShapeDtypeStruct((N_CLASSES, 1), jnp.int32),
    )(prompt)


def kernel(prompt, table):
    idx = jnp.swapaxes(prompt, 0, 1).reshape(-1)
    rows = _sc_gather(table, idx)
    embedding = jnp.swapaxes(rows.reshape(CTX_LEN, N_CLASSES, D_MODEL), 0, 1)
    eos = _tc_argmax(prompt).reshape(N_CLASSES)
    return (embedding, eos)
